# baseline (speedup 1.0000x reference)
import functools

import jax
import jax.numpy as jnp
from jax import lax
from jax.experimental import pallas as pl
from jax.experimental.pallas import tpu as pltpu
from jax.experimental.pallas import tpu_sc as plsc

D = 128
NC = 2
NS = 16
NW = NC * NS
GB = 128
NB = 4
EPS = 1e-5


def _make_sc_gather(num_tokens):
    assert num_tokens % (NW * GB) == 0
    ch = num_tokens // (NW * GB)
    assert ch % 2 == 0 and ch > NB

    mesh = plsc.VectorSubcoreMesh(
        core_axis_name="c", subcore_axis_name="s", num_cores=NC,
        num_subcores=NS)

    @functools.partial(
        pl.kernel,
        mesh=mesh,
        out_type=jax.ShapeDtypeStruct((num_tokens, D), jnp.float32),
        compiler_params=pltpu.CompilerParams(needs_layout_passes=False),
        scratch_types=[
            pltpu.VMEM((ch, GB), jnp.int32),
            pltpu.VMEM((NB, GB, D), jnp.float32),
            pltpu.SemaphoreType.DMA,
            pltpu.SemaphoreType.DMA,
            pltpu.SemaphoreType.DMA,
            pltpu.SemaphoreType.DMA,
            pltpu.SemaphoreType.DMA,
            pltpu.SemaphoreType.DMA,
            pltpu.SemaphoreType.DMA,
            pltpu.SemaphoreType.DMA,
        ],
    )
    def k(table_hbm, idx_hbm, out_hbm, idx_v, buf_v,
          is0, is1, is2, is3, os0, os1, os2, os3):
        wid = lax.axis_index("s") * NC + lax.axis_index("c")
        tok0 = wid * ch * GB

        pltpu.sync_copy(idx_hbm.at[wid], idx_v)

        in_sems = (is0, is1, is2, is3)
        out_sems = (os0, os1, os2, os3)

        def start_in(c, b):
            pltpu.make_async_copy(
                table_hbm.at[idx_v.at[c]], buf_v.at[b], in_sems[b]).start()

        def wait_in(b):
            pltpu.make_async_copy(
                table_hbm.at[idx_v.at[0]], buf_v.at[b], in_sems[b]).wait()

        def start_out(c, b):
            pltpu.make_async_copy(
                buf_v.at[b], out_hbm.at[pl.ds(tok0 + c * GB, GB)],
                out_sems[b]).start()

        def wait_out(b):
            pltpu.make_async_copy(
                buf_v.at[b], out_hbm.at[pl.ds(tok0, GB)], out_sems[b]).wait()

        for t in range(NB):
            start_in(t, t)

        nq = ch // NB
        tail = ch - nq * NB

        def quad_body(q, _):
            c0 = NB * q
            for t in range(NB):
                wait_in(t)
                start_out(c0 + t, t)
            for t in range(NB):
                nxt = c0 + NB + t

                def refill(tt=t, nn=nxt):
                    wait_out(tt)
                    start_in(nn, tt)

                pl.when(nxt < ch)(refill)
            return 0

        lax.fori_loop(0, nq, quad_body, 0)

        for t in range(tail):
            wait_in(t)
            start_out(nq * NB + t, t)
        for t in range(NB):
            wait_out(t)

    return k


def _tc_ln_body(x_ref, g_ref, b_ref, o_ref):
    x = x_ref[...]
    mean = jnp.mean(x, axis=1, keepdims=True)
    xc = x - mean
    var = jnp.mean(xc * xc, axis=1, keepdims=True)
    o_ref[...] = xc * lax.rsqrt(var + EPS) * g_ref[...] + b_ref[...]


def _make_tc_ln(num_tokens, blk):
    assert num_tokens % blk == 0
    return pl.pallas_call(
        _tc_ln_body,
        grid=(num_tokens // blk,),
        in_specs=[
            pl.BlockSpec((blk, D), lambda i: (i, 0)),
            pl.BlockSpec((D,), lambda i: (0,)),
            pl.BlockSpec((D,), lambda i: (0,)),
        ],
        out_specs=pl.BlockSpec((blk, D), lambda i: (i, 0)),
        out_shape=jax.ShapeDtypeStruct((num_tokens, D), jnp.float32),
    )


_sc_gather = _make_sc_gather(1024 * 200)
_tc_ln = _make_tc_ln(1024 * 200, 2048)


@jax.jit
def kernel(x, table, gamma, beta):
    b, s = x.shape
    idx = x.reshape(NW, b * s // (NW * GB), GB).astype(jnp.int32)
    raw = _sc_gather(table, idx)
    out = _tc_ln(raw, gamma, beta)
    return out.reshape(b, s, D)
